# baseline (speedup 1.0000x reference)
import functools
import jax
import jax.numpy as jnp
import numpy as np
from jax import lax
from jax.experimental import pallas as pl
from jax.experimental.pallas import tpu as pltpu
from jax.experimental.pallas import tpu_sc as plsc

_ANCHORS = [1.3221, 1.73145, 3.19275, 4.00944, 5.05587, 8.09892, 9.47112,
            4.84053, 11.2364, 10.0071]
_NA = 5
_NC = 20
_TAU = 0.6
_NB = 16
_NHW = 169
_PPAD = 176
_NCH = 125
_IMG_W = _NCH * _PPAD
_TGT_W = 256
_L = 16
_NW_SC = 32


def _f(v):
    return jnp.full((_L,), v, jnp.float32)


def _i(v):
    return jnp.full((_L,), v, jnp.int32)


def _bcast(ref, idx_vec):
    return plsc.load_gather(ref, [idx_vec])


def _sigmoid(v):
    return 1.0 / (1.0 + jnp.exp(-v))


def _softlog(x):
    bits = lax.bitcast_convert_type(x, jnp.uint32)
    e = ((bits >> 23) & jnp.uint32(255)).astype(jnp.int32) - 127
    m = lax.bitcast_convert_type(
        (bits & jnp.uint32(0x007FFFFF)) | jnp.uint32(0x3F800000), jnp.float32)
    big = m > 1.4142135
    m = jnp.where(big, m * 0.5, m)
    e = e + big.astype(jnp.int32)
    z = (m - 1.0) / (m + 1.0)
    z2 = z * z
    p = z * (2.0 + z2 * (0.6666666666 + z2 * (0.4 + z2 * (
        0.2857142857 + z2 * 0.2222222222))))
    return e.astype(jnp.float32) * 0.6931471805599453 + p


def _hot2(pxl, pxh, pyl, pyh, tau_pa, qxl, qxh, qyl, qyh, tau_sa):
    ow = jnp.minimum(pxh, qxh) - jnp.maximum(pxl, qxl)
    oh = jnp.minimum(pyh, qyh) - jnp.maximum(pyl, qyl)
    carea = ow * oh
    rhs = tau_pa + tau_sa
    return (ow > 0.0) & (oh > 0.0) & (carea * (1.0 + _TAU) > rhs)


def _iou_ref(ax, ay, aw, ah, bx, by, bw, bh):
    mx = jnp.minimum(ax - aw / 2.0, bx - bw / 2.0)
    Mx = jnp.maximum(ax + aw / 2.0, bx + bw / 2.0)
    my = jnp.minimum(ay - ah / 2.0, by - bh / 2.0)
    My = jnp.maximum(ay + ah / 2.0, by + bh / 2.0)
    uw = Mx - mx
    uh = My - my
    cw = aw + bw - uw
    ch = ah + bh - uh
    carea = jnp.where((cw <= 0) | (ch <= 0), 0.0, cw * ch)
    uarea = aw * ah + bw * bh - carea
    return carea / uarea


def _sc_body(out_hbm, tgt_hbm, res_hbm, img, tgt, mxl, mxh, myl, myh, msa,
             bgx, bgy, bgw, bgh, btx, bty, btw, bth, baw, bah, bti, bcell,
             bval, bwin, accb, dsem):
    nc = lax.axis_index("c")
    ns = lax.axis_index("s")
    wid = ns * 2 + nc
    b = wid // 2
    half = wid % 2

    img_cp = pltpu.async_copy(out_hbm.at[pl.ds(b * _IMG_W, _IMG_W)], img, dsem)
    pltpu.sync_copy(tgt_hbm.at[pl.ds(b * _TGT_W, _TGT_W)], tgt)

    lane = lax.iota(jnp.int32, _L)

    carry0 = 0
    for tv in range(4):
        tvec = lane + tv * _L
        k5 = tvec * 5
        cap = _i(_TGT_W - 1)
        tcls = _bcast(tgt, jnp.minimum(k5, cap))
        x = _bcast(tgt, jnp.minimum(k5 + 1, cap))
        y = _bcast(tgt, jnp.minimum(k5 + 2, cap))
        w = _bcast(tgt, jnp.minimum(k5 + 3, cap))
        h = _bcast(tgt, jnp.minimum(k5 + 4, cap))
        zc = (x == 0.0).astype(jnp.int32)
        cs = plsc.cumsum(zc)
        valid = ((cs + carry0) == 0) & (tvec < 50)
        carry0 = carry0 + jnp.sum(zc)

        def mulex(t, c):
            ti = lax.bitcast_convert_type(t, jnp.uint32)
            thi = lax.bitcast_convert_type(ti & jnp.uint32(0xFFFFF000),
                                           jnp.float32)
            tlo = t - thi
            p = t * c
            e = (thi * c - p) + tlo * c
            return p, e

        def floorfrac(p, e):
            g = p.astype(jnp.int32)
            gf = g.astype(jnp.float32)
            d = p - gf
            g = g - ((d == 0.0) & (e < 0.0)).astype(jnp.int32)
            gf = g.astype(jnp.float32)
            return g, (p - gf) + e

        gxp, gxe = mulex(x, 13.0)
        gyp, gye = mulex(y, 13.0)
        gi, txv = floorfrac(gxp, gxe)
        gj, tyv = floorfrac(gyp, gye)
        gw = w * 13.0
        gh = h * 13.0

        bestv = _f(-1.0)
        bestn = _i(0)
        for a in range(_NA):
            aw = _ANCHORS[2 * a]
            ah = _ANCHORS[2 * a + 1]
            aiou = _iou_ref(_f(0.0), _f(0.0), _f(aw), _f(ah),
                            _f(0.0), _f(0.0), gw, gh)
            gt = aiou > bestv
            bestn = jnp.where(gt, _i(a), bestn)
            bestv = jnp.maximum(bestv, aiou)
        awb = _f(_ANCHORS[0])
        ahb = _f(_ANCHORS[1])
        for a in range(1, _NA):
            sel = bestn == a
            awb = jnp.where(sel, _f(_ANCHORS[2 * a]), awb)
            ahb = jnp.where(sel, _f(_ANCHORS[2 * a + 1]), ahb)
        cell = bestn * _NHW + gj * 13 + gi

        sl = pl.ds(tv * _L, _L)
        far = _f(1e9)
        mxl[sl] = jnp.where(valid, gxp - gw * 0.5, far)
        mxh[sl] = jnp.where(valid, gxp + gw * 0.5, far)
        myl[sl] = jnp.where(valid, gyp - gh * 0.5, far)
        myh[sl] = jnp.where(valid, gyp + gh * 0.5, far)
        msa[sl] = jnp.where(valid, _TAU * (gw * gh), _f(0.0))
        bgx[sl] = gxp
        bgy[sl] = gyp
        bgw[sl] = gw
        bgh[sl] = gh
        btx[sl] = txv
        bty[sl] = tyv
        btw[sl] = _softlog(gw / awb)
        bth[sl] = _softlog(gh / ahb)
        baw[sl] = awb
        bah[sl] = ahb
        ti = tcls.astype(jnp.int32)
        bti[sl] = jnp.clip(ti, 0, _NC - 1)
        bcell[sl] = cell
        bval[sl] = valid.astype(jnp.int32)

    img_cp.wait()

    def hot_loop(boxes):
        n = len(boxes)

        def tbody(t, c):
            tq = c[n]
            qxl = _bcast(mxl, tq)
            qxh = _bcast(mxh, tq)
            qyl = _bcast(myl, tq)
            qyh = _bcast(myh, tq)
            sa = _bcast(msa, tq)
            out = []
            for k in range(n):
                pxl, pxh, pyl, pyh, tpa = boxes[k]
                out.append(c[k] | _hot2(pxl, pxh, pyl, pyh, tpa,
                                        qxl, qxh, qyl, qyh, sa))
            out.append(tq + 1)
            return tuple(out)

        init = tuple([lane < 0] * n + [_i(0)])
        res = lax.fori_loop(0, 50, tbody, init)
        return [res[k] for k in range(n)]

    def dense_anchor(an, vlist):
        boxes = []
        for v in vlist:
            base = an * 25 * _PPAD + v * _L
            xr = img[pl.ds(base, _L)]
            yr = img[pl.ds(base + _PPAD, _L)]
            wr = img[pl.ds(base + 2 * _PPAD, _L)]
            hr = img[pl.ds(base + 3 * _PPAD, _L)]
            p0 = v * _L
            gxg = ((lane + p0) % 13).astype(jnp.float32)
            gyg = ((lane + p0) // 13).astype(jnp.float32)
            px = _sigmoid(xr) + gxg
            py = _sigmoid(yr) + gyg
            pw = jnp.exp(wr) * _ANCHORS[2 * an]
            ph = jnp.exp(hr) * _ANCHORS[2 * an + 1]
            hw = pw * 0.5
            hh = ph * 0.5
            boxes.append((px - hw, px + hw, py - hh, py + hh,
                          _TAU * (pw * ph)))
        hots = hot_loop(boxes)
        a = _f(0.0)
        for v, hot in zip(vlist, hots):
            base = an * 25 * _PPAD + v * _L
            cr = img[pl.ds(base + 4 * _PPAD, _L)]
            conf = _sigmoid(cr)
            keep = ~hot
            nvalid = _L if v < 10 else _NHW - 10 * _L
            if nvalid < _L:
                keep = keep & (lane < nvalid)
            a = a + jnp.where(keep, 0.5 * conf * conf, _f(0.0))
        return a

    accb[...] = _f(0.0)

    @pl.when(half == 0)
    def _():
        a = _f(0.0)
        for an in range(_NA):
            a = a + dense_anchor(an, list(range(7)))
        accb[...] = accb[...] + a

    @pl.when(half == 1)
    def _():
        acc1 = _f(0.0)
        for an in range(_NA):
            acc1 = acc1 + dense_anchor(an, list(range(7, 11)))

        def dbody(t, c):
            k0, k1, k2, k3, tq = c
            bc = _bcast(bcell, tq)
            vb = _bcast(bval, tq)
            alive = vb != 0
            ks = []
            for tv, k in enumerate((k0, k1, k2, k3)):
                cv = bcell[pl.ds(tv * _L, _L)]
                tl = lane + tv * _L
                ks.append(k | ((cv == bc) & alive & (tl < tq)))
            return ks[0], ks[1], ks[2], ks[3], tq + 1

        f = lane < 0
        k0, k1, k2, k3, _unused = lax.fori_loop(0, 50, dbody,
                                                (f, f, f, f, _i(0)))
        kills = (k0, k1, k2, k3)
        for tv in range(4):
            sl = pl.ds(tv * _L, _L)
            bwin[sl] = ((bval[sl] != 0) & (~kills[tv])).astype(jnp.int32)

        for tv in range(4):
            sl = pl.ds(tv * _L, _L)
            win = bwin[sl] != 0
            cell = bcell[sl]
            p_ix = cell % _NHW
            a_ix = cell // _NHW
            gi_f = (p_ix % 13).astype(jnp.float32)
            gj_f = (p_ix // 13).astype(jnp.float32)
            fb = a_ix * (25 * _PPAD) + p_ix
            xr = plsc.load_gather(img, [fb])
            yr = plsc.load_gather(img, [fb + _PPAD])
            wr = plsc.load_gather(img, [fb + 2 * _PPAD])
            hr = plsc.load_gather(img, [fb + 3 * _PPAD])
            cr = plsc.load_gather(img, [fb + 4 * _PPAD])
            x_c = _sigmoid(xr)
            y_c = _sigmoid(yr)
            conf_c = _sigmoid(cr)
            awb = baw[sl]
            ahb = bah[sl]
            pxc = x_c + gi_f
            pyc = y_c + gj_f
            pwc = jnp.exp(wr) * awb
            phc = jnp.exp(hr) * ahb
            gx = bgx[sl]
            gy = bgy[sl]
            gw = bgw[sl]
            gh = bgh[sl]
            tconf = _iou_ref(gx, gy, gw, gh, pxc, pyc, pwc, phc)
            hwc = pwc * 0.5
            hhc = phc * 0.5
            hot_c = hot_loop([(pxc - hwc, pxc + hwc, pyc - hhc, pyc + hhc,
                               _TAU * (pwc * phc))])[0]

            dconf = conf_c - tconf
            dx = x_c - btx[sl]
            dy = y_c - bty[sl]
            dw = wr - btw[sl]
            dh = hr - bth[sl]
            confcorr = 2.5 * dconf * dconf - jnp.where(
                hot_c, _f(0.0), 0.5 * conf_c * conf_c)
            coord = 0.5 * (dx * dx + dy * dy + dw * dw + dh * dh)
            cvs = []
            for q in range(_NC):
                cvs.append(plsc.load_gather(img, [fb + (5 + q) * _PPAD]))
            m = cvs[0]
            for q in range(1, _NC):
                m = jnp.maximum(m, cvs[q])
            s = _f(0.0)
            for q in range(_NC):
                s = s + jnp.exp(cvs[q] - m)
            lse = m + _softlog(s)
            picked = plsc.load_gather(img, [fb + (5 + bti[sl]) * _PPAD])
            contrib = confcorr + coord - (picked - lse)
            acc1 = acc1 + jnp.where(win, contrib, _f(0.0))

        accb[...] = accb[...] + acc1

    pltpu.sync_copy(accb, res_hbm.at[pl.ds(wid * _L, _L)])


@jax.jit
def _run(outp, tgtp):
    mesh = plsc.VectorSubcoreMesh(core_axis_name="c", subcore_axis_name="s",
                                  num_cores=2, num_subcores=16)
    fn = pl.kernel(
        _sc_body,
        out_type=jax.ShapeDtypeStruct((_NW_SC * _L,), jnp.float32),
        mesh=mesh,
        compiler_params=pltpu.CompilerParams(needs_layout_passes=False),
        scratch_types=[
            pltpu.VMEM((_IMG_W,), jnp.float32),
            pltpu.VMEM((_TGT_W,), jnp.float32),
        ] + [pltpu.VMEM((64,), jnp.float32) for _ in range(15)] + [
            pltpu.VMEM((64,), jnp.int32) for _ in range(4)
        ] + [pltpu.VMEM((_L,), jnp.float32), pltpu.SemaphoreType.DMA],
    )
    return jnp.sum(fn(outp, tgtp))


def kernel(output, target):
    outp = jnp.pad(output.reshape(_NB, _NCH, _NHW),
                   ((0, 0), (0, 0), (0, _PPAD - _NHW))).reshape(-1)
    tgtp = jnp.pad(target, ((0, 0), (0, _TGT_W - 250))).reshape(-1)
    return _run(outp, tgtp)


# speedup vs baseline: 1.1713x; 1.1713x over previous
import functools
import jax
import jax.numpy as jnp
import numpy as np
from jax import lax
from jax.experimental import pallas as pl
from jax.experimental.pallas import tpu as pltpu
from jax.experimental.pallas import tpu_sc as plsc

_ANCHORS = [1.3221, 1.73145, 3.19275, 4.00944, 5.05587, 8.09892, 9.47112,
            4.84053, 11.2364, 10.0071]
_NA = 5
_NC = 20
_TAU = 0.6
_NB = 16
_NHW = 169
_PPAD = 176
_NCH = 125
_IMG_W = _NCH * _PPAD
_TGT_W = 256
_L = 16
_NW_SC = 32


def _f(v):
    return jnp.full((_L,), v, jnp.float32)


def _i(v):
    return jnp.full((_L,), v, jnp.int32)


def _bcast(ref, idx_vec):
    return plsc.load_gather(ref, [idx_vec])


def _sigmoid(v):
    return 1.0 / (1.0 + jnp.exp(-v))


def _softlog(x):
    bits = lax.bitcast_convert_type(x, jnp.uint32)
    e = ((bits >> 23) & jnp.uint32(255)).astype(jnp.int32) - 127
    m = lax.bitcast_convert_type(
        (bits & jnp.uint32(0x007FFFFF)) | jnp.uint32(0x3F800000), jnp.float32)
    big = m > 1.4142135
    m = jnp.where(big, m * 0.5, m)
    e = e + big.astype(jnp.int32)
    z = (m - 1.0) / (m + 1.0)
    z2 = z * z
    p = z * (2.0 + z2 * (0.6666666666 + z2 * (0.4 + z2 * (
        0.2857142857 + z2 * 0.2222222222))))
    return e.astype(jnp.float32) * 0.6931471805599453 + p


def _hot_score(pxl, pxh, pyl, pyh, tau_pa, qxl, qxh, qyl, qyh, tau_sa):
    ow = jnp.maximum(jnp.minimum(pxh, qxh) - jnp.maximum(pxl, qxl), 0.0)
    oh = jnp.maximum(jnp.minimum(pyh, qyh) - jnp.maximum(pyl, qyl), 0.0)
    return ow * oh - (tau_pa + tau_sa)


def _iou_ref(ax, ay, aw, ah, bx, by, bw, bh):
    mx = jnp.minimum(ax - aw / 2.0, bx - bw / 2.0)
    Mx = jnp.maximum(ax + aw / 2.0, bx + bw / 2.0)
    my = jnp.minimum(ay - ah / 2.0, by - bh / 2.0)
    My = jnp.maximum(ay + ah / 2.0, by + bh / 2.0)
    uw = Mx - mx
    uh = My - my
    cw = aw + bw - uw
    ch = ah + bh - uh
    carea = jnp.where((cw <= 0) | (ch <= 0), 0.0, cw * ch)
    uarea = aw * ah + bw * bh - carea
    return carea / uarea


def _sc_body(out_hbm, tgt_hbm, res_hbm, img, tgt, mxl, mxh, myl, myh, msa,
             bgx, bgy, bgw, bgh, btx, bty, btw, bth, baw, bah, bti, bcell,
             bval, bwin, accb, dsem):
    nc = lax.axis_index("c")
    ns = lax.axis_index("s")
    wid = ns * 2 + nc
    b = wid // 2
    half = wid % 2

    img_cp = pltpu.async_copy(out_hbm.at[pl.ds(b * _IMG_W, _IMG_W)], img, dsem)
    tb = b * 250
    ex2 = tb % 8
    pltpu.sync_copy(tgt_hbm.at[pl.ds(pl.multiple_of(tb - ex2, 8), _TGT_W)],
                    tgt)

    lane = lax.iota(jnp.int32, _L)

    carry0 = 0
    for tv in range(4):
        tvec = lane + tv * _L
        k5 = tvec * 5
        cap = _i(249)
        tcls = _bcast(tgt, jnp.minimum(k5, cap) + ex2)
        x = _bcast(tgt, jnp.minimum(k5 + 1, cap) + ex2)
        y = _bcast(tgt, jnp.minimum(k5 + 2, cap) + ex2)
        w = _bcast(tgt, jnp.minimum(k5 + 3, cap) + ex2)
        h = _bcast(tgt, jnp.minimum(k5 + 4, cap) + ex2)
        zc = (x == 0.0).astype(jnp.int32)
        cs = plsc.cumsum(zc)
        valid = ((cs + carry0) == 0) & (tvec < 50)
        carry0 = carry0 + jnp.sum(zc)

        def mulex(t, c):
            ti = lax.bitcast_convert_type(t, jnp.uint32)
            thi = lax.bitcast_convert_type(ti & jnp.uint32(0xFFFFF000),
                                           jnp.float32)
            tlo = t - thi
            p = t * c
            e = (thi * c - p) + tlo * c
            return p, e

        def floorfrac(p, e):
            g = p.astype(jnp.int32)
            gf = g.astype(jnp.float32)
            d = p - gf
            g = g - ((d == 0.0) & (e < 0.0)).astype(jnp.int32)
            gf = g.astype(jnp.float32)
            return g, (p - gf) + e

        gxp, gxe = mulex(x, 13.0)
        gyp, gye = mulex(y, 13.0)
        gi, txv = floorfrac(gxp, gxe)
        gj, tyv = floorfrac(gyp, gye)
        gw = w * 13.0
        gh = h * 13.0

        bestv = _f(-1.0)
        bestn = _i(0)
        for a in range(_NA):
            aw = _ANCHORS[2 * a]
            ah = _ANCHORS[2 * a + 1]
            aiou = _iou_ref(_f(0.0), _f(0.0), _f(aw), _f(ah),
                            _f(0.0), _f(0.0), gw, gh)
            gt = aiou > bestv
            bestn = jnp.where(gt, _i(a), bestn)
            bestv = jnp.maximum(bestv, aiou)
        awb = _f(_ANCHORS[0])
        ahb = _f(_ANCHORS[1])
        for a in range(1, _NA):
            sel = bestn == a
            awb = jnp.where(sel, _f(_ANCHORS[2 * a]), awb)
            ahb = jnp.where(sel, _f(_ANCHORS[2 * a + 1]), ahb)
        cell = bestn * _NHW + gj * 13 + gi

        sl = pl.ds(tv * _L, _L)
        far = _f(1e9)
        mxl[sl] = jnp.where(valid, gxp - gw * 0.5, far)
        mxh[sl] = jnp.where(valid, gxp + gw * 0.5, far)
        myl[sl] = jnp.where(valid, gyp - gh * 0.5, far)
        myh[sl] = jnp.where(valid, gyp + gh * 0.5, far)
        msa[sl] = jnp.where(valid, (_TAU / (1.0 + _TAU)) * (gw * gh), _f(0.0))
        bgx[sl] = gxp
        bgy[sl] = gyp
        bgw[sl] = gw
        bgh[sl] = gh
        btx[sl] = txv
        bty[sl] = tyv
        btw[sl] = _softlog(gw / awb)
        bth[sl] = _softlog(gh / ahb)
        baw[sl] = awb
        bah[sl] = ahb
        ti = tcls.astype(jnp.int32)
        bti[sl] = jnp.clip(ti, 0, _NC - 1)
        bcell[sl] = cell
        bval[sl] = valid.astype(jnp.int32)

    img_cp.wait()

    def hot_loop(boxes):
        n = len(boxes)

        def tbody(t, c):
            tq = c[n]
            sc = list(c[:n])
            qs = []
            for u in range(5):
                tu = tq + u
                qs.append((_bcast(mxl, tu), _bcast(mxh, tu),
                           _bcast(myl, tu), _bcast(myh, tu),
                           _bcast(msa, tu)))
            for k in range(n):
                pxl, pxh, pyl, pyh, tpa = boxes[k]
                s = sc[k]
                for u in range(5):
                    qxl, qxh, qyl, qyh, sa = qs[u]
                    s = jnp.maximum(s, _hot_score(
                        pxl, pxh, pyl, pyh, tpa, qxl, qxh, qyl, qyh, sa))
                sc[k] = s
            sc.append(tq + 5)
            return tuple(sc)

        init = tuple([_f(-1.0)] * n + [_i(0)])
        res = lax.fori_loop(0, 10, tbody, init)
        return [res[k] > 0.0 for k in range(n)]

    def dense_anchor(an, vlist):
        boxes = []
        for v in vlist:
            base = an * 25 * _PPAD + v * _L
            xr = img[pl.ds(base, _L)]
            yr = img[pl.ds(base + _PPAD, _L)]
            wr = img[pl.ds(base + 2 * _PPAD, _L)]
            hr = img[pl.ds(base + 3 * _PPAD, _L)]
            p0 = v * _L
            gxg = ((lane + p0) % 13).astype(jnp.float32)
            gyg = ((lane + p0) // 13).astype(jnp.float32)
            px = _sigmoid(xr) + gxg
            py = _sigmoid(yr) + gyg
            pw = jnp.exp(wr) * _ANCHORS[2 * an]
            ph = jnp.exp(hr) * _ANCHORS[2 * an + 1]
            hw = pw * 0.5
            hh = ph * 0.5
            boxes.append((px - hw, px + hw, py - hh, py + hh,
                          (_TAU / (1.0 + _TAU)) * (pw * ph)))
        hots = []
        nchunk = 3 if len(boxes) != 4 else 2
        for k0 in range(0, len(boxes), nchunk):
            hots.extend(hot_loop(boxes[k0:k0 + nchunk]))
        a = _f(0.0)
        for v, hot in zip(vlist, hots):
            base = an * 25 * _PPAD + v * _L
            cr = img[pl.ds(base + 4 * _PPAD, _L)]
            conf = _sigmoid(cr)
            keep = ~hot
            nvalid = _L if v < 10 else _NHW - 10 * _L
            if nvalid < _L:
                keep = keep & (lane < nvalid)
            a = a + jnp.where(keep, 0.5 * conf * conf, _f(0.0))
        return a

    accb[...] = _f(0.0)

    @pl.when(half == 0)
    def _():
        a = _f(0.0)
        for an in range(_NA):
            a = a + dense_anchor(an, list(range(6)))
        accb[...] = accb[...] + a

    @pl.when(half == 1)
    def _():
        acc1 = _f(0.0)
        for an in range(_NA):
            acc1 = acc1 + dense_anchor(an, list(range(6, 11)))

        def dbody(t, c):
            k0, k1, k2, k3, tq = c
            bc = _bcast(bcell, tq)
            vb = _bcast(bval, tq)
            alive = vb != 0
            ks = []
            for tv, k in enumerate((k0, k1, k2, k3)):
                cv = bcell[pl.ds(tv * _L, _L)]
                tl = lane + tv * _L
                ks.append(k | ((cv == bc) & alive & (tl < tq)))
            return ks[0], ks[1], ks[2], ks[3], tq + 1

        f = lane < 0
        k0, k1, k2, k3, _unused = lax.fori_loop(0, 50, dbody,
                                                (f, f, f, f, _i(0)))
        kills = (k0, k1, k2, k3)
        for tv in range(4):
            sl = pl.ds(tv * _L, _L)
            bwin[sl] = ((bval[sl] != 0) & (~kills[tv])).astype(jnp.int32)

        for tv in range(4):
            sl = pl.ds(tv * _L, _L)
            win = bwin[sl] != 0
            cell = bcell[sl]
            p_ix = cell % _NHW
            a_ix = cell // _NHW
            gi_f = (p_ix % 13).astype(jnp.float32)
            gj_f = (p_ix // 13).astype(jnp.float32)
            fb = a_ix * (25 * _PPAD) + p_ix
            xr = plsc.load_gather(img, [fb])
            yr = plsc.load_gather(img, [fb + _PPAD])
            wr = plsc.load_gather(img, [fb + 2 * _PPAD])
            hr = plsc.load_gather(img, [fb + 3 * _PPAD])
            cr = plsc.load_gather(img, [fb + 4 * _PPAD])
            x_c = _sigmoid(xr)
            y_c = _sigmoid(yr)
            conf_c = _sigmoid(cr)
            awb = baw[sl]
            ahb = bah[sl]
            pxc = x_c + gi_f
            pyc = y_c + gj_f
            pwc = jnp.exp(wr) * awb
            phc = jnp.exp(hr) * ahb
            gx = bgx[sl]
            gy = bgy[sl]
            gw = bgw[sl]
            gh = bgh[sl]
            tconf = _iou_ref(gx, gy, gw, gh, pxc, pyc, pwc, phc)
            hwc = pwc * 0.5
            hhc = phc * 0.5
            hot_c = hot_loop([(pxc - hwc, pxc + hwc, pyc - hhc, pyc + hhc,
                               (_TAU / (1.0 + _TAU)) * (pwc * phc))])[0]

            dconf = conf_c - tconf
            dx = x_c - btx[sl]
            dy = y_c - bty[sl]
            dw = wr - btw[sl]
            dh = hr - bth[sl]
            confcorr = 2.5 * dconf * dconf - jnp.where(
                hot_c, _f(0.0), 0.5 * conf_c * conf_c)
            coord = 0.5 * (dx * dx + dy * dy + dw * dw + dh * dh)
            cvs = []
            for q in range(_NC):
                cvs.append(plsc.load_gather(img, [fb + (5 + q) * _PPAD]))
            m = cvs[0]
            for q in range(1, _NC):
                m = jnp.maximum(m, cvs[q])
            s = _f(0.0)
            for q in range(_NC):
                s = s + jnp.exp(cvs[q] - m)
            lse = m + _softlog(s)
            picked = plsc.load_gather(img, [fb + (5 + bti[sl]) * _PPAD])
            contrib = confcorr + coord - (picked - lse)
            acc1 = acc1 + jnp.where(win, contrib, _f(0.0))

        accb[...] = accb[...] + acc1

    pltpu.sync_copy(accb, res_hbm.at[pl.ds(wid * _L, _L)])


@jax.jit
def _run(outp, tgtp):
    mesh = plsc.VectorSubcoreMesh(core_axis_name="c", subcore_axis_name="s",
                                  num_cores=2, num_subcores=16)
    fn = pl.kernel(
        _sc_body,
        out_type=jax.ShapeDtypeStruct((_NW_SC * _L,), jnp.float32),
        mesh=mesh,
        compiler_params=pltpu.CompilerParams(needs_layout_passes=False),
        scratch_types=[
            pltpu.VMEM((_IMG_W,), jnp.float32),
            pltpu.VMEM((_TGT_W,), jnp.float32),
        ] + [pltpu.VMEM((64,), jnp.float32) for _ in range(15)] + [
            pltpu.VMEM((64,), jnp.int32) for _ in range(4)
        ] + [pltpu.VMEM((_L,), jnp.float32), pltpu.SemaphoreType.DMA],
    )
    return jnp.sum(fn(outp, tgtp))


def kernel(output, target):
    outp = jnp.pad(output.reshape(_NB, _NCH, _NHW),
                   ((0, 0), (0, 0), (0, _PPAD - _NHW))).reshape(-1)
    return _run(outp, target.reshape(-1))
